# baseline (speedup 1.0000x reference)
import functools

import jax
import jax.numpy as jnp
from jax import lax
from jax.experimental import pallas as pl
from jax.experimental.pallas import tpu as pltpu
from jax.experimental.pallas import tpu_sc as plsc

N = 10000
E = 320000
D = 128

NC = 2
NS = 16
NW = NC * NS
L = 16

CHUNK = 64
NCHUNK0 = 225
NCHUNK1 = 89
G = CHUNK // L
E_TILE0 = CHUNK * NCHUNK0
E_TILE1 = CHUNK * NCHUNK1
E_PAD = NS * (E_TILE0 + E_TILE1)
TRASH = N

ACC_ROWS = 10240
ROWS_PER_TILE = ACC_ROWS // NS

R = 4

_mesh = plsc.VectorSubcoreMesh(core_axis_name="c", subcore_axis_name="s")


@functools.partial(
    pl.kernel,
    out_type=[
        jax.ShapeDtypeStruct((NC * ACC_ROWS, D), jnp.float32),
        jax.ShapeDtypeStruct((NW * ACC_ROWS,), jnp.float32),
    ],
    mesh=_mesh,
    compiler_params=pltpu.CompilerParams(needs_layout_passes=False),
    scratch_types=[
        pltpu.VMEM((CHUNK,), jnp.int32),
        pltpu.VMEM((CHUNK,), jnp.int32),
        pltpu.VMEM((CHUNK,), jnp.int32),
        pltpu.VMEM((CHUNK,), jnp.int32),
        pltpu.VMEM((CHUNK,), jnp.int32),
        pltpu.VMEM((CHUNK,), jnp.int32),
        pltpu.VMEM((CHUNK,), jnp.int32),
        pltpu.VMEM((CHUNK,), jnp.int32),
        pltpu.VMEM((CHUNK,), jnp.float32),
        pltpu.VMEM((CHUNK,), jnp.float32),
        pltpu.VMEM((CHUNK,), jnp.float32),
        pltpu.VMEM((CHUNK,), jnp.float32),
        pltpu.VMEM((CHUNK, D), jnp.float32),
        pltpu.VMEM((CHUNK, D), jnp.float32),
        pltpu.VMEM((CHUNK, D), jnp.float32),
        pltpu.VMEM((CHUNK, D), jnp.float32),
        pltpu.VMEM((CHUNK,), jnp.int32),
        pltpu.VMEM((CHUNK,), jnp.int32),
        pltpu.VMEM((CHUNK,), jnp.int32),
        pltpu.VMEM((ACC_ROWS,), jnp.float32),
        pltpu.VMEM_SHARED((ACC_ROWS, D), jnp.float32),
        pltpu.SemaphoreType.DMA,
        pltpu.SemaphoreType.DMA,
        pltpu.SemaphoreType.DMA,
        pltpu.SemaphoreType.DMA,
        pltpu.SemaphoreType.DMA,
        pltpu.SemaphoreType.DMA,
        pltpu.SemaphoreType.DMA,
        pltpu.SemaphoreType.DMA,
        pltpu.SemaphoreType.DMA,
        pltpu.SemaphoreType.DMA,
        pltpu.SemaphoreType.DMA,
        pltpu.SemaphoreType.DMA,
    ],
)
def _scatter_accum(h_hbm, src_hbm, dst_hbm, w_hbm, zeros_hbm,
                   zflat_hbm, parts_hbm, cnts_hbm,
                   si0, si1, si2, si3, di0, di1, di2, di3,
                   wv0, wv1, wv2, wv3, rw0, rw1, rw2, rw3,
                   sx0, sx1, tidx, hist, accum,
                   is0, is1, is2, is3, gs0, gs1, gs2, gs3,
                   ss0, ss1, ss2, ss3):
    c = lax.axis_index("c")
    s = lax.axis_index("s")
    wid = s * NC + c
    ebase = c * (NS * E_TILE0) + s * jnp.where(c == 0, E_TILE0, E_TILE1)
    nchunk = jnp.where(c == 0, NCHUNK0, NCHUNK1)
    ntrip = jnp.where(c == 0, (NCHUNK0 - 1) // R, (NCHUNK1 - 1) // R)

    SI = (si0, si1, si2, si3)
    DI = (di0, di1, di2, di3)
    WV = (wv0, wv1, wv2, wv3)
    ROWS = (rw0, rw1, rw2, rw3)
    SX = (sx0, sx1)
    ISEM = (is0, is1, is2, is3)
    GSEM = (gs0, gs1, gs2, gs3)
    SSEM = (ss0, ss1, ss2, ss3)

    ones16 = jnp.ones((L,), jnp.float32)
    trash16 = jnp.full((L,), TRASH, jnp.int32)

    def issue_idx(j, q):
        eb = ebase + j * CHUNK
        pltpu.async_copy(src_hbm.at[pl.ds(eb, CHUNK)], SI[q], ISEM[q])
        pltpu.async_copy(dst_hbm.at[pl.ds(eb, CHUNK)], DI[q], ISEM[q])
        pltpu.async_copy(w_hbm.at[pl.ds(eb, CHUNK)], WV[q], ISEM[q])

    def wait_idx(q):
        pltpu.make_async_copy(src_hbm.at[pl.ds(0, CHUNK)], SI[q], ISEM[q]).wait()
        pltpu.make_async_copy(dst_hbm.at[pl.ds(0, CHUNK)], DI[q], ISEM[q]).wait()
        pltpu.make_async_copy(w_hbm.at[pl.ds(0, CHUNK)], WV[q], ISEM[q]).wait()

    def issue_gather(q):
        pltpu.async_copy(h_hbm.at[SI[q]], ROWS[q], GSEM[q])

    def wait_gather(q):
        pltpu.make_async_copy(h_hbm.at[SI[q]], ROWS[q], GSEM[q]).wait()

    def issue_scatter(q, p):
        pltpu.async_copy(ROWS[q], accum.at[SX[p]], SSEM[q], add=True)

    def wait_scatter(q, p):
        pltpu.make_async_copy(ROWS[q], accum.at[SX[p]], SSEM[q]).wait()

    def compute(q, p):
        rows_q, wv_q, di_q = ROWS[q], WV[q], DI[q]
        for g in range(G):
            sl = pl.ds(g * L, L)
            SX[p][sl] = di_q[sl]

        def grp(g, carry):
            w16 = wv_q[pl.ds(g * L, L)]
            d16 = di_q[pl.ds(g * L, L)]
            plsc.addupdate_scatter(hist, [d16], ones16)
            e0 = g * L
            for i in range(L):
                wb = jnp.broadcast_to(w16[i], (L,))
                for k in range(D // L):
                    sl = pl.ds(k * L, L)
                    rows_q[e0 + i, sl] = rows_q[e0 + i, sl] * wb
            return carry

        lax.fori_loop(0, G, grp, 0)

    pltpu.sync_copy(zeros_hbm,
                    accum.at[pl.ds(s * ROWS_PER_TILE, ROWS_PER_TILE)])
    pltpu.sync_copy(zflat_hbm, hist)
    for g in range(G):
        tidx[pl.ds(g * L, L)] = trash16
    plsc.subcore_barrier()

    issue_idx(0, 0)
    issue_idx(1, 1)
    issue_idx(2, 2)
    pltpu.async_copy(rw2, accum.at[tidx], ss2, add=True)
    pltpu.async_copy(rw3, accum.at[tidx], ss3, add=True)
    wait_idx(0)
    issue_gather(0)
    wait_idx(1)
    issue_gather(1)

    def sub_body(j, q):
        q1 = (q + 1) % R
        q2 = (q + 2) % R
        q3 = (q + 3) % R
        p = q % 2
        wait_gather(q)
        wait_scatter(q2, p)
        wait_idx(q2)
        issue_gather(q2)
        issue_idx(jnp.minimum(j + 3, nchunk - 1), q3)
        compute(q, p)
        issue_scatter(q, p)

    def body4(t, carry):
        j = 4 * t
        sub_body(j, 0)
        sub_body(j + 1, 1)
        sub_body(j + 2, 2)
        sub_body(j + 3, 3)
        return carry

    lax.fori_loop(0, ntrip, body4, 0)

    wait_gather(0)
    wait_scatter(2, 0)
    compute(0, 0)
    issue_scatter(0, 0)
    wait_scatter(3, 1)
    wait_scatter(0, 0)
    wait_gather(1)
    wait_idx(2)

    plsc.subcore_barrier()

    r0 = s * ROWS_PER_TILE
    pltpu.sync_copy(
        accum.at[pl.ds(r0, ROWS_PER_TILE)],
        parts_hbm.at[pl.ds(c * ACC_ROWS + r0, ROWS_PER_TILE)])
    pltpu.sync_copy(hist, cnts_hbm.at[pl.ds(wid * ACC_ROWS, ACC_ROWS)])


_BLK = 400


def _finalize_body(h_ref, p_ref, c_ref, o_ref):
    p = p_ref[...]
    cnt = jnp.sum(c_ref[...], axis=1, keepdims=True)
    ssum = p[0] + p[1]
    h_n = ssum / jnp.maximum(cnt, 1.0)
    o_ref[:, :D] = h_ref[...]
    o_ref[:, D:] = h_n


def kernel(h, edge_index, edge_w):
    src = edge_index[0]
    dst = edge_index[1]
    pad = E_PAD - E
    src_p = jnp.concatenate([src, jnp.zeros((pad,), jnp.int32)])
    dst_p = jnp.concatenate([dst, jnp.full((pad,), TRASH, jnp.int32)])
    w_p = jnp.concatenate([edge_w, jnp.zeros((pad,), jnp.float32)])
    zeros = jnp.zeros((ROWS_PER_TILE, D), jnp.float32)
    zflat = jnp.zeros((ACC_ROWS,), jnp.float32)

    parts, cnts = _scatter_accum(h, src_p, dst_p, w_p, zeros, zflat)
    parts = parts.reshape(NC, ACC_ROWS, D)
    cnts = cnts.reshape(NW, ACC_ROWS).T

    out = pl.pallas_call(
        _finalize_body,
        out_shape=jax.ShapeDtypeStruct((N, 2 * D), jnp.float32),
        grid=(N // _BLK,),
        in_specs=[
            pl.BlockSpec((_BLK, D), lambda i: (i, 0)),
            pl.BlockSpec((NC, _BLK, D), lambda i: (0, i, 0)),
            pl.BlockSpec((_BLK, NW), lambda i: (i, 0)),
        ],
        out_specs=pl.BlockSpec((_BLK, 2 * D), lambda i: (i, 0)),
    )(h, parts, cnts)
    return out


# speedup vs baseline: 1.0347x; 1.0347x over previous
import functools

import jax
import jax.numpy as jnp
from jax import lax
from jax.experimental import pallas as pl
from jax.experimental.pallas import tpu as pltpu
from jax.experimental.pallas import tpu_sc as plsc

N = 10000
E = 320000
D = 128

NC = 2
NS = 16
NW = NC * NS
L = 16

CHUNK = 64
NCHUNK0 = 213
NCHUNK1 = 101
G = CHUNK // L
E_TILE0 = CHUNK * NCHUNK0
E_TILE1 = CHUNK * NCHUNK1
E_PAD = NS * (E_TILE0 + E_TILE1)
TRASH = N

ACC_ROWS = 10240
ROWS_PER_TILE = ACC_ROWS // NS

R = 4

_mesh = plsc.VectorSubcoreMesh(core_axis_name="c", subcore_axis_name="s")


@functools.partial(
    pl.kernel,
    out_type=[
        jax.ShapeDtypeStruct((NC * ACC_ROWS, D), jnp.float32),
        jax.ShapeDtypeStruct((NW * ACC_ROWS,), jnp.float32),
    ],
    mesh=_mesh,
    compiler_params=pltpu.CompilerParams(needs_layout_passes=False),
    scratch_types=[
        pltpu.VMEM((CHUNK,), jnp.int32),
        pltpu.VMEM((CHUNK,), jnp.int32),
        pltpu.VMEM((CHUNK,), jnp.int32),
        pltpu.VMEM((CHUNK,), jnp.int32),
        pltpu.VMEM((CHUNK,), jnp.int32),
        pltpu.VMEM((CHUNK,), jnp.int32),
        pltpu.VMEM((CHUNK,), jnp.int32),
        pltpu.VMEM((CHUNK,), jnp.int32),
        pltpu.VMEM((CHUNK,), jnp.float32),
        pltpu.VMEM((CHUNK,), jnp.float32),
        pltpu.VMEM((CHUNK,), jnp.float32),
        pltpu.VMEM((CHUNK,), jnp.float32),
        pltpu.VMEM((CHUNK, D), jnp.float32),
        pltpu.VMEM((CHUNK, D), jnp.float32),
        pltpu.VMEM((CHUNK, D), jnp.float32),
        pltpu.VMEM((CHUNK, D), jnp.float32),
        pltpu.VMEM((CHUNK,), jnp.int32),
        pltpu.VMEM((CHUNK,), jnp.int32),
        pltpu.VMEM((CHUNK,), jnp.int32),
        pltpu.VMEM((ACC_ROWS,), jnp.float32),
        pltpu.VMEM_SHARED((ACC_ROWS, D), jnp.float32),
        pltpu.SemaphoreType.DMA,
        pltpu.SemaphoreType.DMA,
        pltpu.SemaphoreType.DMA,
        pltpu.SemaphoreType.DMA,
        pltpu.SemaphoreType.DMA,
        pltpu.SemaphoreType.DMA,
        pltpu.SemaphoreType.DMA,
        pltpu.SemaphoreType.DMA,
        pltpu.SemaphoreType.DMA,
        pltpu.SemaphoreType.DMA,
        pltpu.SemaphoreType.DMA,
        pltpu.SemaphoreType.DMA,
    ],
)
def _scatter_accum(h_hbm, src_hbm, dst_hbm, w_hbm, zeros_hbm,
                   zflat_hbm, parts_hbm, cnts_hbm,
                   si0, si1, si2, si3, di0, di1, di2, di3,
                   wv0, wv1, wv2, wv3, rw0, rw1, rw2, rw3,
                   sx0, sx1, tidx, hist, accum,
                   is0, is1, is2, is3, gs0, gs1, gs2, gs3,
                   ss0, ss1, ss2, ss3):
    c = lax.axis_index("c")
    s = lax.axis_index("s")
    wid = s * NC + c
    ebase = c * (NS * E_TILE0) + s * jnp.where(c == 0, E_TILE0, E_TILE1)
    nchunk = jnp.where(c == 0, NCHUNK0, NCHUNK1)
    ntrip = jnp.where(c == 0, (NCHUNK0 - 1) // R, (NCHUNK1 - 1) // R)

    SI = (si0, si1, si2, si3)
    DI = (di0, di1, di2, di3)
    WV = (wv0, wv1, wv2, wv3)
    ROWS = (rw0, rw1, rw2, rw3)
    SX = (sx0, sx1)
    ISEM = (is0, is1, is2, is3)
    GSEM = (gs0, gs1, gs2, gs3)
    SSEM = (ss0, ss1, ss2, ss3)

    ones16 = jnp.ones((L,), jnp.float32)
    trash16 = jnp.full((L,), TRASH, jnp.int32)

    def issue_idx(j, q):
        eb = ebase + j * CHUNK
        pltpu.async_copy(src_hbm.at[pl.ds(eb, CHUNK)], SI[q], ISEM[q])
        pltpu.async_copy(dst_hbm.at[pl.ds(eb, CHUNK)], DI[q], ISEM[q])
        pltpu.async_copy(w_hbm.at[pl.ds(eb, CHUNK)], WV[q], ISEM[q])

    def wait_idx(q):
        pltpu.make_async_copy(src_hbm.at[pl.ds(0, CHUNK)], SI[q], ISEM[q]).wait()
        pltpu.make_async_copy(dst_hbm.at[pl.ds(0, CHUNK)], DI[q], ISEM[q]).wait()
        pltpu.make_async_copy(w_hbm.at[pl.ds(0, CHUNK)], WV[q], ISEM[q]).wait()

    def issue_gather(q):
        pltpu.async_copy(h_hbm.at[SI[q]], ROWS[q], GSEM[q])

    def wait_gather(q):
        pltpu.make_async_copy(h_hbm.at[SI[q]], ROWS[q], GSEM[q]).wait()

    def issue_scatter(q, p):
        pltpu.async_copy(ROWS[q], accum.at[SX[p]], SSEM[q], add=True)

    def wait_scatter(q, p):
        pltpu.make_async_copy(ROWS[q], accum.at[SX[p]], SSEM[q]).wait()

    def compute(q, p):
        rows_q, wv_q, di_q = ROWS[q], WV[q], DI[q]
        for g in range(G):
            sl = pl.ds(g * L, L)
            SX[p][sl] = di_q[sl]

        def grp(g, carry):
            w16 = wv_q[pl.ds(g * L, L)]
            d16 = di_q[pl.ds(g * L, L)]
            plsc.addupdate_scatter(hist, [d16], ones16)
            e0 = g * L
            for i in range(L):
                wb = jnp.broadcast_to(w16[i], (L,))
                for k in range(D // L):
                    sl = pl.ds(k * L, L)
                    rows_q[e0 + i, sl] = rows_q[e0 + i, sl] * wb
            return carry

        lax.fori_loop(0, G, grp, 0)

    pltpu.sync_copy(zeros_hbm,
                    accum.at[pl.ds(s * ROWS_PER_TILE, ROWS_PER_TILE)])
    pltpu.sync_copy(zflat_hbm, hist)
    for g in range(G):
        tidx[pl.ds(g * L, L)] = trash16
    plsc.subcore_barrier()

    issue_idx(0, 0)
    issue_idx(1, 1)
    issue_idx(2, 2)
    pltpu.async_copy(rw2, accum.at[tidx], ss2, add=True)
    pltpu.async_copy(rw3, accum.at[tidx], ss3, add=True)
    wait_idx(0)
    issue_gather(0)
    wait_idx(1)
    issue_gather(1)

    def sub_body(j, q):
        q1 = (q + 1) % R
        q2 = (q + 2) % R
        q3 = (q + 3) % R
        p = q % 2
        wait_gather(q)
        wait_scatter(q2, p)
        wait_idx(q2)
        issue_gather(q2)
        issue_idx(jnp.minimum(j + 3, nchunk - 1), q3)
        compute(q, p)
        issue_scatter(q, p)

    def body4(t, carry):
        j = 4 * t
        sub_body(j, 0)
        sub_body(j + 1, 1)
        sub_body(j + 2, 2)
        sub_body(j + 3, 3)
        return carry

    lax.fori_loop(0, ntrip, body4, 0)

    wait_gather(0)
    wait_scatter(2, 0)
    compute(0, 0)
    issue_scatter(0, 0)
    wait_scatter(3, 1)
    wait_scatter(0, 0)
    wait_gather(1)
    wait_idx(2)

    plsc.subcore_barrier()

    r0 = s * ROWS_PER_TILE
    pltpu.sync_copy(
        accum.at[pl.ds(r0, ROWS_PER_TILE)],
        parts_hbm.at[pl.ds(c * ACC_ROWS + r0, ROWS_PER_TILE)])
    pltpu.sync_copy(hist, cnts_hbm.at[pl.ds(wid * ACC_ROWS, ACC_ROWS)])


_BLK = 400


def _finalize_body(h_ref, p_ref, c_ref, o_ref):
    p = p_ref[...]
    cnt = jnp.sum(c_ref[...], axis=1, keepdims=True)
    ssum = p[0] + p[1]
    h_n = ssum / jnp.maximum(cnt, 1.0)
    o_ref[:, :D] = h_ref[...]
    o_ref[:, D:] = h_n


def kernel(h, edge_index, edge_w):
    src = edge_index[0]
    dst = edge_index[1]
    pad = E_PAD - E
    src_p = jnp.concatenate([src, jnp.zeros((pad,), jnp.int32)])
    dst_p = jnp.concatenate([dst, jnp.full((pad,), TRASH, jnp.int32)])
    w_p = jnp.concatenate([edge_w, jnp.zeros((pad,), jnp.float32)])
    zeros = jnp.zeros((ROWS_PER_TILE, D), jnp.float32)
    zflat = jnp.zeros((ACC_ROWS,), jnp.float32)

    parts, cnts = _scatter_accum(h, src_p, dst_p, w_p, zeros, zflat)
    parts = parts.reshape(NC, ACC_ROWS, D)
    cnts = cnts.reshape(NW, ACC_ROWS).T

    out = pl.pallas_call(
        _finalize_body,
        out_shape=jax.ShapeDtypeStruct((N, 2 * D), jnp.float32),
        grid=(N // _BLK,),
        in_specs=[
            pl.BlockSpec((_BLK, D), lambda i: (i, 0)),
            pl.BlockSpec((NC, _BLK, D), lambda i: (0, i, 0)),
            pl.BlockSpec((_BLK, NW), lambda i: (i, 0)),
        ],
        out_specs=pl.BlockSpec((_BLK, 2 * D), lambda i: (i, 0)),
    )(h, parts, cnts)
    return out
